# baseline (speedup 1.0000x reference)
import functools

import jax
import jax.numpy as jnp
from jax.experimental import pallas as pl
from jax.experimental.pallas import tpu as pltpu

_K, _V, _H, _CW = 5, 8, 8, 64
_C1, _CDR, _NK = 32, 4, 16
_O_W1 = 0
_O_WDR = _O_W1 + _K * _V
_O_W2 = _O_WDR + _CW
_O_WIHF = _O_W2 + _K * _CW
_O_WIHR = _O_WIHF + _CW
_O_WHH = _O_WIHR + _CW


def _blockdiag2(w):
    z = jnp.zeros_like(w)
    return jnp.concatenate(
        [jnp.concatenate([w, z], axis=1), jnp.concatenate([z, w], axis=1)],
        axis=0)


def _gate_interleave(w):
    r = w.shape[0]
    w4 = w.reshape(r, 4, 16)
    z4 = jnp.zeros_like(w4)
    top = jnp.stack([w4, z4], axis=2).reshape(r, 128)
    bot = jnp.stack([z4, w4], axis=2).reshape(r, 128)
    return jnp.concatenate([top, bot], axis=0)


def _body(L, TB, Lp, tok_ref, w1s_ref, wdr_ref, w2s_ref, wih_ref, whh_ref,
          vv_ref, out_ref):
    f32 = jnp.float32
    bf16 = jnp.bfloat16
    N = TB * L
    L1 = L - (_K - 1)
    PAD = (_K - 1) // 2

    tok = tok_ref[...]
    lane = jax.lax.broadcasted_iota(jnp.int32, (N, 2 * _V), 1)
    tsel = jnp.where(lane < _V, tok[:, 0:1], tok[:, 1:2])
    onehot = (tsel == (lane & (_V - 1))).astype(f32)

    oh_p = jnp.concatenate(
        [onehot, jnp.zeros(((_K - 1) * TB, 2 * _V), f32)], axis=0)

    acc = None
    for k in range(_K):
        p = jnp.dot(oh_p[k * TB:k * TB + N, :], w1s_ref[k],
                    preferred_element_type=f32)
        acc = p if acc is None else acc + p
    h1 = jnp.maximum(acc + vv_ref[0:1, :], 0.0)

    hdr = jnp.dot(h1, wdr_ref[...], preferred_element_type=f32) \
        + vv_ref[1:2, :]
    row = jax.lax.broadcasted_iota(jnp.int32, (N, 1), 0)
    hdr = jnp.where(row < L1 * TB, hdr, 0.0)
    z2 = jnp.zeros((PAD * TB, 2 * _CW), f32)
    hdr_p = jnp.concatenate([z2, hdr, z2], axis=0)

    acc2 = None
    for k in range(_K):
        p = jnp.dot(hdr_p[k * TB:k * TB + N, :], w2s_ref[k],
                    preferred_element_type=f32)
        acc2 = p if acc2 is None else acc2 + p
    h2 = (jnp.maximum(acc2 + vv_ref[2:3, :], 0.0)
          * vv_ref[3:4, :] + vv_ref[4:5, :])

    M3 = (3 * (Lp - 1) + 1) * TB
    m = jnp.maximum(jnp.maximum(h2[0:M3, :], h2[TB:M3 + TB, :]),
                    h2[2 * TB:M3 + 2 * TB, :])
    pf = jnp.concatenate(
        [m[3 * t * TB:(3 * t + 1) * TB, :] for t in range(Lp)], axis=0)
    pr = jnp.concatenate(
        [m[3 * (Lp - 1 - t) * TB:(3 * (Lp - 1 - t) + 1) * TB, :]
         for t in range(Lp)], axis=0)

    g_in = (jnp.dot(pf, wih_ref[0], preferred_element_type=f32)
            + jnp.dot(pr, wih_ref[1], preferred_element_type=f32)
            + vv_ref[5:6, :])

    G = 2 * 2 * _H
    gi = [g_in[:, q * G:(q + 1) * G] for q in range(4)]
    hcat = jnp.zeros((TB, G), f32)
    ccat = jnp.zeros((TB, G), f32)
    for t in range(Lp):
        s = slice(t * TB, (t + 1) * TB)
        ai, af, ao, ag = (
            gi[q][s, :] + jnp.dot(hcat, whh_ref[q],
                                  preferred_element_type=f32)
            for q in range(4))
        ccat = (jax.nn.sigmoid(af) * ccat
                + jax.nn.sigmoid(ai) * jnp.tanh(ag))
        hcat = jax.nn.sigmoid(ao) * jnp.tanh(ccat)

    out_ref[0:TB, :] = hcat[:, 0:2 * _H]
    out_ref[TB:2 * TB, :] = hcat[:, 2 * _H:4 * _H]


@jax.jit
def kernel(x_tokens, wmat, vvec):
    B, L = x_tokens.shape
    L1 = L - (_K - 1)
    Lp = L1 // 3
    assert L1 >= 1 and Lp >= 1

    TB = 128
    ST = 2 * TB
    Bp = -(-B // ST) * ST
    grid = Bp // ST
    N = TB * L

    tok = jnp.asarray(x_tokens, jnp.int32)
    if Bp != B:
        tok = jnp.pad(tok, ((0, Bp - B), (0, 0)))
    tok2 = (tok.reshape(grid, 2, TB, L).transpose(0, 3, 2, 1)
            .reshape(grid * N, 2))

    w1s = jnp.stack(
        [_blockdiag2(wmat[_O_W1 + k * _V:_O_W1 + (k + 1) * _V, :])
         for k in range(_K)])
    wdr2 = _blockdiag2(wmat[_O_WDR:_O_WDR + _CW, :])
    w2s = jnp.stack(
        [_blockdiag2(wmat[_O_W2 + k * _CW:_O_W2 + (k + 1) * _CW, :])
         for k in range(_K)])
    wih2 = jnp.stack(
        [_gate_interleave(wmat[_O_WIHF:_O_WIHF + _CW, :]),
         _gate_interleave(wmat[_O_WIHR:_O_WIHR + _CW, :])])
    whh_gi = _gate_interleave(wmat[_O_WHH:_O_WHH + 2 * _H, :])
    whh2 = jnp.stack([whh_gi[:, 32 * q:32 * (q + 1)]
                      for q in range(4)])

    def dup(row):
        v = vvec[row:row + 1, :]
        return jnp.concatenate([v, v], axis=1)

    bc4 = vvec[5:6, :].reshape(1, 4, 2 * _H)
    bcat2 = jnp.stack([bc4, bc4], axis=2).reshape(1, 128)
    vv2 = jnp.concatenate(
        [dup(0), dup(1), dup(2), dup(3), dup(4),
         bcat2, jnp.zeros((2, 128), jnp.float32)], axis=0)

    body = functools.partial(_body, L, TB, Lp)
    out = pl.pallas_call(
        body,
        out_shape=jax.ShapeDtypeStruct((Bp, 2 * _H), jnp.float32),
        grid=(grid,),
        in_specs=[
            pl.BlockSpec((N, 2), lambda i: (i, 0)),
            pl.BlockSpec(w1s.shape, lambda i: (0, 0, 0)),
            pl.BlockSpec(wdr2.shape, lambda i: (0, 0)),
            pl.BlockSpec(w2s.shape, lambda i: (0, 0, 0)),
            pl.BlockSpec(wih2.shape, lambda i: (0, 0, 0)),
            pl.BlockSpec(whh2.shape, lambda i: (0, 0, 0)),
            pl.BlockSpec(vv2.shape, lambda i: (0, 0)),
        ],
        out_specs=pl.BlockSpec((ST, 2 * _H), lambda i: (i, 0)),
        compiler_params=pltpu.CompilerParams(
            dimension_semantics=("parallel",)),
    )(tok2, w1s, wdr2, w2s, wih2, whh2, vv2)
    return out[:B]


# speedup vs baseline: 1.2529x; 1.2529x over previous
import functools

import jax
import jax.numpy as jnp
from jax.experimental import pallas as pl
from jax.experimental.pallas import tpu as pltpu

_K, _V, _H, _CW = 5, 8, 8, 64
_C1, _CDR, _NK = 32, 4, 16
_O_W1 = 0
_O_WDR = _O_W1 + _K * _V
_O_W2 = _O_WDR + _CW
_O_WIHF = _O_W2 + _K * _CW
_O_WIHR = _O_WIHF + _CW
_O_WHH = _O_WIHR + _CW


def _blockdiag2(w):
    z = jnp.zeros_like(w)
    return jnp.concatenate(
        [jnp.concatenate([w, z], axis=1), jnp.concatenate([z, w], axis=1)],
        axis=0)


def _gate_interleave(w):
    r = w.shape[0]
    w4 = w.reshape(r, 4, 16)
    z4 = jnp.zeros_like(w4)
    top = jnp.stack([w4, z4], axis=2).reshape(r, 128)
    bot = jnp.stack([z4, w4], axis=2).reshape(r, 128)
    return jnp.concatenate([top, bot], axis=0)


def _body(L, TB, Lp, tok_ref, w1s_ref, wdr_ref, w2s_ref, wih_ref, whh_ref,
          vv_ref, out_ref):
    f32 = jnp.float32
    bf16 = jnp.bfloat16
    N = TB * L
    L1 = L - (_K - 1)
    PAD = (_K - 1) // 2

    tok = tok_ref[...]
    lane = jax.lax.broadcasted_iota(jnp.int32, (N, 2 * _V), 1)
    tsel = jnp.where(lane < _V, tok[:, 0:1], tok[:, 1:2])
    onehot = (tsel == (lane & (_V - 1))).astype(f32)

    oh_p = jnp.concatenate(
        [onehot, jnp.zeros(((_K - 1) * TB, 2 * _V), f32)], axis=0)

    acc = None
    for k in range(_K):
        p = jnp.dot(oh_p[k * TB:k * TB + N, :], w1s_ref[k],
                    preferred_element_type=f32)
        acc = p if acc is None else acc + p
    h1 = jnp.maximum(acc + vv_ref[0:1, :], 0.0)

    hdr = jnp.dot(h1, wdr_ref[...], preferred_element_type=f32) \
        + vv_ref[1:2, :]
    row = jax.lax.broadcasted_iota(jnp.int32, (N, 1), 0)
    hdr = jnp.where(row < L1 * TB, hdr, 0.0)
    z2 = jnp.zeros((PAD * TB, 2 * _CW), f32)
    hdr_p = jnp.concatenate([z2, hdr, z2], axis=0)

    acc2 = None
    for k in range(_K):
        p = jnp.dot(hdr_p[k * TB:k * TB + N, :], w2s_ref[k],
                    preferred_element_type=f32)
        acc2 = p if acc2 is None else acc2 + p
    h2 = (jnp.maximum(acc2 + vv_ref[2:3, :], 0.0)
          * vv_ref[3:4, :] + vv_ref[4:5, :])

    M3 = (3 * (Lp - 1) + 1) * TB
    m = jnp.maximum(jnp.maximum(h2[0:M3, :], h2[TB:M3 + TB, :]),
                    h2[2 * TB:M3 + 2 * TB, :])
    pf = jnp.concatenate(
        [m[3 * t * TB:(3 * t + 1) * TB, :] for t in range(Lp)], axis=0)
    pr = jnp.concatenate(
        [m[3 * (Lp - 1 - t) * TB:(3 * (Lp - 1 - t) + 1) * TB, :]
         for t in range(Lp)], axis=0)

    g_in = (jnp.dot(pf, wih_ref[0], preferred_element_type=f32)
            + jnp.dot(pr, wih_ref[1], preferred_element_type=f32)
            + vv_ref[5:6, :])

    G = 2 * 2 * _H
    hcat = jnp.zeros((TB, G), f32)
    ccat = jnp.zeros((TB, G), f32)
    for t in range(Lp):
        gates = g_in[t * TB:(t + 1) * TB, :] + jnp.dot(
            hcat, whh_ref[...], preferred_element_type=f32)
        sig = jax.nn.sigmoid(gates[:, 0:3 * G])
        g = jnp.tanh(gates[:, 3 * G:4 * G])
        ccat = sig[:, G:2 * G] * ccat + sig[:, 0:G] * g
        hcat = sig[:, 2 * G:3 * G] * jnp.tanh(ccat)

    out_ref[0:TB, :] = hcat[:, 0:2 * _H]
    out_ref[TB:2 * TB, :] = hcat[:, 2 * _H:4 * _H]


@jax.jit
def kernel(x_tokens, wmat, vvec):
    B, L = x_tokens.shape
    L1 = L - (_K - 1)
    Lp = L1 // 3
    assert L1 >= 1 and Lp >= 1

    TB = 256
    ST = 2 * TB
    Bp = -(-B // ST) * ST
    grid = Bp // ST
    N = TB * L

    tok = jnp.asarray(x_tokens, jnp.int32)
    if Bp != B:
        tok = jnp.pad(tok, ((0, Bp - B), (0, 0)))
    tok2 = (tok.reshape(grid, 2, TB, L).transpose(0, 3, 2, 1)
            .reshape(grid * N, 2))

    w1s = jnp.stack(
        [_blockdiag2(wmat[_O_W1 + k * _V:_O_W1 + (k + 1) * _V, :])
         for k in range(_K)])
    wdr2 = _blockdiag2(wmat[_O_WDR:_O_WDR + _CW, :])
    w2s = jnp.stack(
        [_blockdiag2(wmat[_O_W2 + k * _CW:_O_W2 + (k + 1) * _CW, :])
         for k in range(_K)])
    wih2 = jnp.stack(
        [_gate_interleave(wmat[_O_WIHF:_O_WIHF + _CW, :]),
         _gate_interleave(wmat[_O_WIHR:_O_WIHR + _CW, :])])
    whh2 = _gate_interleave(wmat[_O_WHH:_O_WHH + 2 * _H, :])

    def dup(row):
        v = vvec[row:row + 1, :]
        return jnp.concatenate([v, v], axis=1)

    bc4 = vvec[5:6, :].reshape(1, 4, 2 * _H)
    bcat2 = jnp.stack([bc4, bc4], axis=2).reshape(1, 128)
    vv2 = jnp.concatenate(
        [dup(0), dup(1), dup(2), dup(3), dup(4),
         bcat2, jnp.zeros((2, 128), jnp.float32)], axis=0)

    body = functools.partial(_body, L, TB, Lp)
    out = pl.pallas_call(
        body,
        out_shape=jax.ShapeDtypeStruct((Bp, 2 * _H), jnp.float32),
        grid=(grid,),
        in_specs=[
            pl.BlockSpec((N, 2), lambda i: (i, 0)),
            pl.BlockSpec(w1s.shape, lambda i: (0, 0, 0)),
            pl.BlockSpec(wdr2.shape, lambda i: (0, 0)),
            pl.BlockSpec(w2s.shape, lambda i: (0, 0, 0)),
            pl.BlockSpec(wih2.shape, lambda i: (0, 0, 0)),
            pl.BlockSpec(whh2.shape, lambda i: (0, 0)),
            pl.BlockSpec(vv2.shape, lambda i: (0, 0)),
        ],
        out_specs=pl.BlockSpec((ST, 2 * _H), lambda i: (i, 0)),
        compiler_params=pltpu.CompilerParams(
            dimension_semantics=("parallel",)),
    )(tok2, w1s, wdr2, w2s, wih2, whh2, vv2)
    return out[:B]


# speedup vs baseline: 1.3162x; 1.0505x over previous
import functools

import jax
import jax.numpy as jnp
from jax.experimental import pallas as pl
from jax.experimental.pallas import tpu as pltpu

_K, _V, _H, _CW = 5, 8, 8, 64
_C1, _CDR, _NK = 32, 4, 16
_O_W1 = 0
_O_WDR = _O_W1 + _K * _V
_O_W2 = _O_WDR + _CW
_O_WIHF = _O_W2 + _K * _CW
_O_WIHR = _O_WIHF + _CW
_O_WHH = _O_WIHR + _CW


def _blockdiag2(w):
    z = jnp.zeros_like(w)
    return jnp.concatenate(
        [jnp.concatenate([w, z], axis=1), jnp.concatenate([z, w], axis=1)],
        axis=0)


def _gate_interleave(w):
    r = w.shape[0]
    w4 = w.reshape(r, 4, 16)
    z4 = jnp.zeros_like(w4)
    top = jnp.stack([w4, z4], axis=2).reshape(r, 128)
    bot = jnp.stack([z4, w4], axis=2).reshape(r, 128)
    return jnp.concatenate([top, bot], axis=0)


def _body(L, TB, Lp, tok_ref, w1s_ref, wdr_ref, w2s_ref, wih_ref, whh_ref,
          vv_ref, out_ref):
    f32 = jnp.float32
    bf16 = jnp.bfloat16
    N = TB * L
    L1 = L - (_K - 1)
    PAD = (_K - 1) // 2

    tok = tok_ref[...]
    lane = jax.lax.broadcasted_iota(jnp.int32, (N, 2 * _V), 1)
    tsel = jnp.where(lane < _V, tok[:, 0:1], tok[:, 1:2])
    onehot = (tsel == (lane & (_V - 1))).astype(f32)

    oh_p = jnp.concatenate(
        [onehot, jnp.zeros(((_K - 1) * TB, 2 * _V), f32)], axis=0)

    acc = None
    for k in range(_K):
        p = jnp.dot(oh_p[k * TB:k * TB + N, :], w1s_ref[k],
                    preferred_element_type=f32)
        acc = p if acc is None else acc + p
    h1 = jnp.maximum(acc + vv_ref[0:1, :], 0.0)

    hdr = jnp.dot(h1, wdr_ref[...], preferred_element_type=f32) \
        + vv_ref[1:2, :]
    row = jax.lax.broadcasted_iota(jnp.int32, (N, 1), 0)
    hdr = jnp.where(row < L1 * TB, hdr, 0.0)
    z2 = jnp.zeros((PAD * TB, 2 * _CW), f32)
    hdr_p = jnp.concatenate([z2, hdr, z2], axis=0)

    acc2 = None
    for k in range(_K):
        p = jnp.dot(hdr_p[k * TB:k * TB + N, :], w2s_ref[k],
                    preferred_element_type=f32)
        acc2 = p if acc2 is None else acc2 + p
    h2 = (jnp.maximum(acc2 + vv_ref[2:3, :], 0.0)
          * vv_ref[3:4, :] + vv_ref[4:5, :])

    M3 = (3 * (Lp - 1) + 1) * TB
    m = jnp.maximum(jnp.maximum(h2[0:M3, :], h2[TB:M3 + TB, :]),
                    h2[2 * TB:M3 + 2 * TB, :])
    pf = jnp.concatenate(
        [m[3 * t * TB:(3 * t + 1) * TB, :] for t in range(Lp)], axis=0)
    pr = jnp.concatenate(
        [m[3 * (Lp - 1 - t) * TB:(3 * (Lp - 1 - t) + 1) * TB, :]
         for t in range(Lp)], axis=0)

    g_in = (jnp.dot(pf, wih_ref[0], preferred_element_type=f32)
            + jnp.dot(pr, wih_ref[1], preferred_element_type=f32)
            + vv_ref[5:6, :])

    G = 2 * 2 * _H
    hcat = jnp.zeros((TB, G), f32)
    ccat = jnp.zeros((TB, G), f32)
    for t in range(Lp):
        gates = g_in[t * TB:(t + 1) * TB, :] + jnp.dot(
            hcat, whh_ref[...], preferred_element_type=f32)
        sig = jax.nn.sigmoid(gates[:, 0:3 * G])
        g = jnp.tanh(gates[:, 3 * G:4 * G])
        ccat = sig[:, G:2 * G] * ccat + sig[:, 0:G] * g
        hcat = sig[:, 2 * G:3 * G] * jnp.tanh(ccat)

    out_ref[0:TB, :] = hcat[:, 0:2 * _H]
    out_ref[TB:2 * TB, :] = hcat[:, 2 * _H:4 * _H]


@jax.jit
def kernel(x_tokens, wmat, vvec):
    B, L = x_tokens.shape
    L1 = L - (_K - 1)
    Lp = L1 // 3
    assert L1 >= 1 and Lp >= 1

    TB = 384
    ST = 2 * TB
    Bp = -(-B // ST) * ST
    grid = Bp // ST
    N = TB * L

    tok = jnp.asarray(x_tokens, jnp.int32)
    if Bp != B:
        tok = jnp.pad(tok, ((0, Bp - B), (0, 0)))
    tok2 = (tok.reshape(grid, 2, TB, L).transpose(0, 3, 2, 1)
            .reshape(grid * N, 2))

    w1s = jnp.stack(
        [_blockdiag2(wmat[_O_W1 + k * _V:_O_W1 + (k + 1) * _V, :])
         for k in range(_K)])
    wdr2 = _blockdiag2(wmat[_O_WDR:_O_WDR + _CW, :])
    w2s = jnp.stack(
        [_blockdiag2(wmat[_O_W2 + k * _CW:_O_W2 + (k + 1) * _CW, :])
         for k in range(_K)])
    wih2 = jnp.stack(
        [_gate_interleave(wmat[_O_WIHF:_O_WIHF + _CW, :]),
         _gate_interleave(wmat[_O_WIHR:_O_WIHR + _CW, :])])
    whh2 = _gate_interleave(wmat[_O_WHH:_O_WHH + 2 * _H, :])

    def dup(row):
        v = vvec[row:row + 1, :]
        return jnp.concatenate([v, v], axis=1)

    bc4 = vvec[5:6, :].reshape(1, 4, 2 * _H)
    bcat2 = jnp.stack([bc4, bc4], axis=2).reshape(1, 128)
    vv2 = jnp.concatenate(
        [dup(0), dup(1), dup(2), dup(3), dup(4),
         bcat2, jnp.zeros((2, 128), jnp.float32)], axis=0)

    body = functools.partial(_body, L, TB, Lp)
    out = pl.pallas_call(
        body,
        out_shape=jax.ShapeDtypeStruct((Bp, 2 * _H), jnp.float32),
        grid=(grid,),
        in_specs=[
            pl.BlockSpec((N, 2), lambda i: (i, 0)),
            pl.BlockSpec(w1s.shape, lambda i: (0, 0, 0)),
            pl.BlockSpec(wdr2.shape, lambda i: (0, 0)),
            pl.BlockSpec(w2s.shape, lambda i: (0, 0, 0)),
            pl.BlockSpec(wih2.shape, lambda i: (0, 0, 0)),
            pl.BlockSpec(whh2.shape, lambda i: (0, 0)),
            pl.BlockSpec(vv2.shape, lambda i: (0, 0)),
        ],
        out_specs=pl.BlockSpec((ST, 2 * _H), lambda i: (i, 0)),
        compiler_params=pltpu.CompilerParams(
            dimension_semantics=("parallel",)),
    )(tok2, w1s, wdr2, w2s, wih2, whh2, vv2)
    return out[:B]
